# baseline (speedup 1.0000x reference)
import functools
from math import ceil

import jax
import jax.numpy as jnp
from jax import lax
from jax.experimental import pallas as pl
from jax.experimental.pallas import tpu as pltpu
from jax.experimental.pallas import tpu_sc as plsc

DT = 1.0
TIME = 4.0
DTSQ = DT * DT

NC = 2
NS = 16
NW = NC * NS
L = 16
K = 96
F1 = 0.5


def _mm_body(x_ref, w0_ref, b0_ref, w1_ref, b1_ref, p0_ref, p1_ref):
    xb = x_ref[...]
    dn = (((1,), (1,)), ((), ()))
    p0_ref[...] = lax.dot_general(xb, w0_ref[...], dn,
                                  preferred_element_type=jnp.float32) + b0_ref[...]
    p1_ref[...] = lax.dot_general(xb, w1_ref[...], dn,
                                  preferred_element_type=jnp.float32) + b1_ref[...]


def _matmuls(x, W0, b0, W1, b1, br):
    n, d = x.shape
    grid = (n // br,)
    return pl.pallas_call(
        _mm_body,
        grid=grid,
        in_specs=[
            pl.BlockSpec((br, d), lambda i: (i, 0)),
            pl.BlockSpec((d, d), lambda i: (0, 0)),
            pl.BlockSpec((1, d), lambda i: (0, 0)),
            pl.BlockSpec((d, d), lambda i: (0, 0)),
            pl.BlockSpec((1, d), lambda i: (0, 0)),
        ],
        out_specs=[
            pl.BlockSpec((br, d), lambda i: (i, 0)),
            pl.BlockSpec((br, d), lambda i: (i, 0)),
        ],
        out_shape=[
            jax.ShapeDtypeStruct((n, d), jnp.float32),
            jax.ShapeDtypeStruct((n, d), jnp.float32),
        ],
    )(x, W0, b0.reshape(1, -1), W1, b1.reshape(1, -1))


def _make_stats(n, e, e_pad, ntrash):
    epw = e_pad // NW
    mesh = plsc.VectorSubcoreMesh(core_axis_name="c", subcore_axis_name="s")

    @functools.partial(
        pl.kernel,
        mesh=mesh,
        out_type=(
            jax.ShapeDtypeStruct((NW, 1, n), jnp.float32),
            jax.ShapeDtypeStruct((NW, 1, n), jnp.float32),
            jax.ShapeDtypeStruct((e_pad,), jnp.int32),
            jax.ShapeDtypeStruct((e_pad,), jnp.int32),
        ),
        scratch_types=[
            pltpu.VMEM((epw,), jnp.int32),
            pltpu.VMEM((epw,), jnp.int32),
            pltpu.VMEM((epw,), jnp.int32),
            pltpu.VMEM((epw,), jnp.int32),
            pltpu.VMEM((1, n), jnp.float32),
            pltpu.VMEM((1, n), jnp.float32),
        ],
        compiler_params=pltpu.CompilerParams(needs_layout_passes=False),
    )
    def stats(row_hbm, col_hbm, degp, selfp, ridx, cidx,
              row_v, col_v, rout_v, cout_v, deg_v, self_v):
        c = lax.axis_index("c")
        s = lax.axis_index("s")
        wid = c * NS + s
        base = wid * epw
        slast = e - (NW - 1) * epw

        @pl.when(wid < NW - 1)
        def _():
            pltpu.sync_copy(row_hbm.at[pl.ds(base, epw)], row_v)
            pltpu.sync_copy(col_hbm.at[pl.ds(base, epw)], col_v)

        @pl.when(wid == NW - 1)
        def _():
            pltpu.sync_copy(row_hbm.at[pl.ds(base, slast)],
                            row_v.at[pl.ds(0, slast)])
            pltpu.sync_copy(col_hbm.at[pl.ds(base, slast)],
                            col_v.at[pl.ds(0, slast)])

        zero16 = jnp.zeros((L,), jnp.float32)

        def zbody(i, carry):
            deg_v[0, pl.ds(i * L, L)] = zero16
            self_v[0, pl.ds(i * L, L)] = zero16
            return carry

        lax.fori_loop(0, n // L, zbody, 0)

        ones16 = jnp.ones((L,), jnp.float32)
        iota16 = lax.iota(jnp.int32, L)
        zidx = jnp.zeros((L,), jnp.int32)

        def body(i, carry):
            off = i * L
            r = row_v[pl.ds(off, L)]
            cc = col_v[pl.ds(off, L)]
            eid = base + off + iota16
            real = eid < e
            nself = r != cc
            plsc.addupdate_scatter(deg_v, [zidx, cc], ones16, mask=real & nself)
            plsc.addupdate_scatter(self_v, [zidx, r], ones16, mask=real & (~nself))
            rout_v[pl.ds(off, L)] = jnp.where(real, r, lax.rem(eid, n))
            trash = n + lax.rem(eid, ntrash)
            cout_v[pl.ds(off, L)] = jnp.where(real & nself, cc, trash)
            return carry

        lax.fori_loop(0, epw // L, body, 0)

        pltpu.sync_copy(deg_v, degp.at[wid])
        pltpu.sync_copy(self_v, selfp.at[wid])
        pltpu.sync_copy(rout_v, ridx.at[pl.ds(base, epw)])
        pltpu.sync_copy(cout_v, cidx.at[pl.ds(base, epw)])

    return stats


def _prep_body(degp_ref, selfp_ref, phi0_ref, dis_ref, c2_ref, featp_ref):
    degsum = jnp.sum(degp_ref[...], axis=(0, 1))
    selfsum = jnp.sum(selfp_ref[...], axis=(0, 1))
    loop_w = (selfsum > 0).astype(jnp.float32)
    deg = degsum + loop_w
    pos = deg > 0
    dis = jnp.where(pos, lax.rsqrt(deg), 0.0)
    c2 = DTSQ * loop_w * jnp.where(pos, 1.0 / deg, 1.0) + 2.0
    dis_ref[...] = dis[:, None]
    c2_ref[...] = c2[:, None]
    featp_ref[...] = dis[:, None] * phi0_ref[...]


def _prep(degp, selfp, phi0, br):
    n, d = phi0.shape
    del br
    return pl.pallas_call(
        _prep_body,
        out_shape=[
            jax.ShapeDtypeStruct((n, 1), jnp.float32),
            jax.ShapeDtypeStruct((n, 1), jnp.float32),
            jax.ShapeDtypeStruct((n, d), jnp.float32),
        ],
    )(degp, selfp, phi0)


def _make_gather(n, d, cpt0, cpt1):
    wpt = ceil(ceil(n / NS) / 8) * 8
    npad = NS * wpt
    mesh = plsc.VectorSubcoreMesh(core_axis_name="c", subcore_axis_name="s")

    @functools.partial(
        pl.kernel,
        mesh=mesh,
        out_type=jax.ShapeDtypeStruct((NC, npad, d), jnp.float32),
        scratch_types=[
            pltpu.VMEM_SHARED((npad, d), jnp.float32),
            pltpu.VMEM((max(cpt0, cpt1) * K,), jnp.int32),
            pltpu.VMEM((1, K), jnp.int32),
            pltpu.VMEM((1, K), jnp.int32),
            pltpu.VMEM((1, K), jnp.int32),
            pltpu.VMEM((K, d), jnp.float32),
            pltpu.VMEM((K, d), jnp.float32),
            pltpu.VMEM((K, d), jnp.float32),
        ] + [pltpu.SemaphoreType.DMA] * 9,
        compiler_params=pltpu.CompilerParams(needs_layout_passes=False),
    )
    def gather(featp, ridx, cidx, parts, acc_sh, ri_t, cb0, cb1, cb2,
               r0, r1, r2, gs0, gs1, gs2, ss0, ss1, ss2, is0, is1, is2):
        c = lax.axis_index("c")
        s = lax.axis_index("s")

        zero16 = jnp.zeros((L,), jnp.float32)

        def zbody(i, carry):
            r0[i // (d // L), pl.ds((i % (d // L)) * L, L)] = zero16
            return carry

        lax.fori_loop(0, K * (d // L), zbody, 0)
        row0 = s * wpt
        nfull = wpt // K
        rem = wpt % K
        for j in range(nfull):
            pltpu.sync_copy(r0, acc_sh.at[pl.ds(row0 + j * K, K)])
        if rem:
            pltpu.sync_copy(r0.at[pl.ds(0, rem)],
                            acc_sh.at[pl.ds(row0 + nfull * K, rem)])
        plsc.subcore_barrier()

        def run(chb, cpt):
            pltpu.sync_copy(ridx.at[pl.ds(chb * K, cpt * K)],
                            ri_t.at[pl.ds(0, cpt * K)])
            rr = (r0, r1, r2)
            cbr = (cb0, cb1, cb2)
            gsr = (gs0, gs1, gs2)
            ssr = (ss0, ss1, ss2)
            isr = (is0, is1, is2)
            for j in range(2):
                pltpu.async_copy(cidx.at[pl.ds((chb + j) * K, K)], cbr[j].at[0],
                                 isr[j])
                pltpu.async_copy(featp.at[ri_t.at[pl.ds(j * K, K)]], rr[j],
                                 gsr[j])

            def step(i, p):
                t = (p + 2) % 3
                cur, cbc, gs_c, ss_c, is_c = rr[p], cbr[p], gsr[p], ssr[p], isr[p]
                tgt, cbt, gs_t, ss_t, is_t = rr[t], cbr[t], gsr[t], ssr[t], isr[t]

                @pl.when(i >= 1)
                def _():
                    pltpu.make_async_copy(tgt, acc_sh.at[cbt.at[0]], ss_t).wait()

                @pl.when(i + 2 < cpt)
                def _():
                    pltpu.async_copy(cidx.at[pl.ds((chb + i + 2) * K, K)],
                                     cbt.at[0], is_t)
                    pltpu.async_copy(featp.at[ri_t.at[pl.ds((i + 2) * K, K)]],
                                     tgt, gs_t)

                pltpu.make_async_copy(featp.at[ri_t.at[pl.ds(i * K, K)]],
                                      cur, gs_c).wait()
                pltpu.make_async_copy(cidx.at[pl.ds((chb + i) * K, K)],
                                      cbc.at[0], is_c).wait()
                pltpu.async_copy(cur, acc_sh.at[cbc.at[0]], ss_c, add=True)

            def body(i, carry):
                for p in range(3):
                    @pl.when(lax.rem(i, 3) == p)
                    def _(i=i, p=p):
                        step(i, p)

                return carry

            lax.fori_loop(0, cpt, body, 0)
            q = cpt - 1
            pltpu.make_async_copy(rr[q % 3], acc_sh.at[cbr[q % 3].at[0]],
                                  ssr[q % 3]).wait()

        @pl.when(c == 1)
        def _():
            run(s * cpt1, cpt1)

        @pl.when(c == 0)
        def _():
            run(NS * cpt1 + s * cpt0, cpt0)

        plsc.subcore_barrier()

        pltpu.sync_copy(acc_sh.at[pl.ds(row0, wpt)], parts.at[c].at[pl.ds(row0, wpt)])

    return gather


def _combine_body(step1, last, parts_ref, f_ref, g_ref, dis_ref, c2_ref, *out_refs):
    pb = parts_ref[...]
    agg = pb[0] + pb[1]
    dis = dis_ref[...]
    zf = DTSQ * dis * agg + c2_ref[...] * f_ref[...]
    if step1:
        xn = DT * g_ref[...] + 0.5 * zf
    else:
        xn = zf - g_ref[...]
    out_refs[0][...] = xn
    if not last:
        out_refs[1][...] = dis * xn


def _combine(parts, f, g, dis, c2, br, step1, last):
    n, d = f.shape
    grid = (n // br,)
    nouts = 1 if last else 2
    return pl.pallas_call(
        functools.partial(_combine_body, step1, last),
        grid=grid,
        in_specs=[
            pl.BlockSpec((NC, br, d), lambda i: (0, i, 0)),
            pl.BlockSpec((br, d), lambda i: (i, 0)),
            pl.BlockSpec((br, d), lambda i: (i, 0)),
            pl.BlockSpec((br, 1), lambda i: (i, 0)),
            pl.BlockSpec((br, 1), lambda i: (i, 0)),
        ],
        out_specs=[pl.BlockSpec((br, d), lambda i: (i, 0))] * nouts,
        out_shape=[jax.ShapeDtypeStruct((n, d), jnp.float32)] * nouts,
    )(parts, f, g, dis, c2)


def kernel(x, edge_index, W0, b0, W1, b1):
    n, d = x.shape
    e = edge_index.shape[1]
    br = 1000

    ct_min = ceil(e / K)
    cpt1 = max(1, round(ct_min * F1 / NS))
    cpt0 = max(1, ceil((ct_min - NS * cpt1) / NS))
    e_pad = NS * (cpt0 + cpt1) * K

    wpt = ceil(ceil(n / NS) / 8) * 8
    ntrash = NS * wpt - n

    phi0, phi1 = _matmuls(x, W0, b0, W1, b1, br)
    degp, selfp, ridx, cidx = _make_stats(n, e, e_pad, ntrash)(edge_index[0], edge_index[1])
    dis, c2, featp = _prep(degp, selfp, phi0, br)

    gather = _make_gather(n, d, cpt0, cpt1)

    n_steps = ceil(TIME / DT)
    parts = gather(featp, ridx, cidx)
    xc, fp = _combine(parts, phi0, phi1, dis, c2, 400, step1=True, last=False)
    xp = phi0
    for k in range(n_steps - 1):
        parts = gather(fp, ridx, cidx)
        if k < n_steps - 2:
            xn, fp = _combine(parts, xc, xp, dis, c2, 400, step1=False, last=False)
        else:
            (xn,) = _combine(parts, xc, xp, dis, c2, 400, step1=False, last=True)
        xp, xc = xc, xn
    return xc


# speedup vs baseline: 1.0728x; 1.0728x over previous
import functools
from math import ceil

import jax
import jax.numpy as jnp
from jax import lax
from jax.experimental import pallas as pl
from jax.experimental.pallas import tpu as pltpu
from jax.experimental.pallas import tpu_sc as plsc

DT = 1.0
TIME = 4.0
DTSQ = DT * DT

NC = 2
NS = 16
NW = NC * NS
L = 16
K = 96
F1 = 0.5


def _mm_body(x_ref, w0_ref, b0_ref, w1_ref, b1_ref, p0_ref, p1_ref):
    xb = x_ref[...]
    dn = (((1,), (1,)), ((), ()))
    p0_ref[...] = lax.dot_general(xb, w0_ref[...], dn,
                                  preferred_element_type=jnp.float32) + b0_ref[...]
    p1_ref[...] = lax.dot_general(xb, w1_ref[...], dn,
                                  preferred_element_type=jnp.float32) + b1_ref[...]


def _matmuls(x, W0, b0, W1, b1, br):
    n, d = x.shape
    grid = (n // br,)
    return pl.pallas_call(
        _mm_body,
        grid=grid,
        in_specs=[
            pl.BlockSpec((br, d), lambda i: (i, 0)),
            pl.BlockSpec((d, d), lambda i: (0, 0)),
            pl.BlockSpec((1, d), lambda i: (0, 0)),
            pl.BlockSpec((d, d), lambda i: (0, 0)),
            pl.BlockSpec((1, d), lambda i: (0, 0)),
        ],
        out_specs=[
            pl.BlockSpec((br, d), lambda i: (i, 0)),
            pl.BlockSpec((br, d), lambda i: (i, 0)),
        ],
        out_shape=[
            jax.ShapeDtypeStruct((n, d), jnp.float32),
            jax.ShapeDtypeStruct((n, d), jnp.float32),
        ],
    )(x, W0, b0.reshape(1, -1), W1, b1.reshape(1, -1))


def _make_stats(n, e, e_pad, ntrash):
    epw = e_pad // NW
    mesh = plsc.VectorSubcoreMesh(core_axis_name="c", subcore_axis_name="s")

    @functools.partial(
        pl.kernel,
        mesh=mesh,
        out_type=(
            jax.ShapeDtypeStruct((NW, 1, n), jnp.float32),
            jax.ShapeDtypeStruct((NW, 1, n), jnp.float32),
            jax.ShapeDtypeStruct((e_pad,), jnp.int32),
            jax.ShapeDtypeStruct((e_pad,), jnp.int32),
        ),
        scratch_types=[
            pltpu.VMEM((epw,), jnp.int32),
            pltpu.VMEM((epw,), jnp.int32),
            pltpu.VMEM((epw,), jnp.int32),
            pltpu.VMEM((epw,), jnp.int32),
            pltpu.VMEM((1, n), jnp.float32),
            pltpu.VMEM((1, n), jnp.float32),
        ],
        compiler_params=pltpu.CompilerParams(needs_layout_passes=False),
    )
    def stats(row_hbm, col_hbm, degp, selfp, ridx, cidx,
              row_v, col_v, rout_v, cout_v, deg_v, self_v):
        c = lax.axis_index("c")
        s = lax.axis_index("s")
        wid = c * NS + s
        base = wid * epw
        slast = e - (NW - 1) * epw

        @pl.when(wid < NW - 1)
        def _():
            pltpu.sync_copy(row_hbm.at[pl.ds(base, epw)], row_v)
            pltpu.sync_copy(col_hbm.at[pl.ds(base, epw)], col_v)

        @pl.when(wid == NW - 1)
        def _():
            pltpu.sync_copy(row_hbm.at[pl.ds(base, slast)],
                            row_v.at[pl.ds(0, slast)])
            pltpu.sync_copy(col_hbm.at[pl.ds(base, slast)],
                            col_v.at[pl.ds(0, slast)])

        zero16 = jnp.zeros((L,), jnp.float32)

        def zbody(i, carry):
            deg_v[0, pl.ds(i * L, L)] = zero16
            self_v[0, pl.ds(i * L, L)] = zero16
            return carry

        lax.fori_loop(0, n // L, zbody, 0)

        ones16 = jnp.ones((L,), jnp.float32)
        iota16 = lax.iota(jnp.int32, L)
        zidx = jnp.zeros((L,), jnp.int32)

        def body(i, carry):
            off = i * L
            r = row_v[pl.ds(off, L)]
            cc = col_v[pl.ds(off, L)]
            eid = base + off + iota16
            real = eid < e
            nself = r != cc
            plsc.addupdate_scatter(deg_v, [zidx, cc], ones16, mask=real & nself)
            plsc.addupdate_scatter(self_v, [zidx, r], ones16, mask=real & (~nself))
            rout_v[pl.ds(off, L)] = jnp.where(real, r, lax.rem(eid, n))
            trash = n + lax.rem(eid, ntrash)
            cout_v[pl.ds(off, L)] = jnp.where(real & nself, cc, trash)
            return carry

        lax.fori_loop(0, epw // L, body, 0)

        pltpu.sync_copy(deg_v, degp.at[wid])
        pltpu.sync_copy(self_v, selfp.at[wid])
        pltpu.sync_copy(rout_v, ridx.at[pl.ds(base, epw)])
        pltpu.sync_copy(cout_v, cidx.at[pl.ds(base, epw)])

    return stats


def _prep_body(degp_ref, selfp_ref, phi0_ref, dis_ref, c2_ref, featp_ref):
    degsum = jnp.sum(degp_ref[...], axis=(0, 1))
    selfsum = jnp.sum(selfp_ref[...], axis=(0, 1))
    loop_w = (selfsum > 0).astype(jnp.float32)
    deg = degsum + loop_w
    pos = deg > 0
    dis = jnp.where(pos, lax.rsqrt(deg), 0.0)
    c2 = DTSQ * loop_w * jnp.where(pos, 1.0 / deg, 1.0) + 2.0
    dis_ref[...] = dis[:, None]
    c2_ref[...] = c2[:, None]
    featp_ref[...] = dis[:, None] * phi0_ref[...]


def _prep(degp, selfp, phi0, br):
    n, d = phi0.shape
    del br
    return pl.pallas_call(
        _prep_body,
        out_shape=[
            jax.ShapeDtypeStruct((n, 1), jnp.float32),
            jax.ShapeDtypeStruct((n, 1), jnp.float32),
            jax.ShapeDtypeStruct((n, d), jnp.float32),
        ],
    )(degp, selfp, phi0)


def _make_gather(n, d, cpt0, cpt1):
    wpt = ceil(ceil(n / NS) / 8) * 8
    npad = NS * wpt
    mesh = plsc.VectorSubcoreMesh(core_axis_name="c", subcore_axis_name="s")

    @functools.partial(
        pl.kernel,
        mesh=mesh,
        out_type=jax.ShapeDtypeStruct((NC, npad, d), jnp.float32),
        scratch_types=[
            pltpu.VMEM_SHARED((npad, d), jnp.float32),
            pltpu.VMEM((max(cpt0, cpt1) * K,), jnp.int32),
            pltpu.VMEM((1, K), jnp.int32),
            pltpu.VMEM((1, K), jnp.int32),
            pltpu.VMEM((1, K), jnp.int32),
            pltpu.VMEM((K, d), jnp.float32),
            pltpu.VMEM((K, d), jnp.float32),
            pltpu.VMEM((K, d), jnp.float32),
        ] + [pltpu.SemaphoreType.DMA] * 9,
        compiler_params=pltpu.CompilerParams(needs_layout_passes=False),
    )
    def gather(featp, ridx, cidx, parts, acc_sh, ri_t, cb0, cb1, cb2,
               r0, r1, r2, gs0, gs1, gs2, ss0, ss1, ss2, is0, is1, is2):
        c = lax.axis_index("c")
        s = lax.axis_index("s")

        zero16 = jnp.zeros((L,), jnp.float32)

        def zbody(i, carry):
            r0[i // (d // L), pl.ds((i % (d // L)) * L, L)] = zero16
            return carry

        lax.fori_loop(0, K * (d // L), zbody, 0)
        row0 = s * wpt
        nfull = wpt // K
        rem = wpt % K
        for j in range(nfull):
            pltpu.sync_copy(r0, acc_sh.at[pl.ds(row0 + j * K, K)])
        if rem:
            pltpu.sync_copy(r0.at[pl.ds(0, rem)],
                            acc_sh.at[pl.ds(row0 + nfull * K, rem)])
        plsc.subcore_barrier()

        def run(chb, cpt):
            pltpu.sync_copy(ridx.at[pl.ds(chb * K, cpt * K)],
                            ri_t.at[pl.ds(0, cpt * K)])
            rr = (r0, r1, r2)
            cbr = (cb0, cb1, cb2)
            gsr = (gs0, gs1, gs2)
            ssr = (ss0, ss1, ss2)
            isr = (is0, is1, is2)
            for j in range(2):
                pltpu.async_copy(cidx.at[pl.ds((chb + j) * K, K)], cbr[j].at[0],
                                 isr[j])
                pltpu.async_copy(featp.at[ri_t.at[pl.ds(j * K, K)]], rr[j],
                                 gsr[j])

            def step(i, p):
                t = (p + 2) % 3
                cur, cbc, gs_c, ss_c, is_c = rr[p], cbr[p], gsr[p], ssr[p], isr[p]
                tgt, cbt, gs_t, ss_t, is_t = rr[t], cbr[t], gsr[t], ssr[t], isr[t]

                @pl.when(i >= 1)
                def _():
                    pltpu.make_async_copy(tgt, acc_sh.at[cbt.at[0]], ss_t).wait()

                @pl.when(i + 2 < cpt)
                def _():
                    pltpu.async_copy(cidx.at[pl.ds((chb + i + 2) * K, K)],
                                     cbt.at[0], is_t)
                    pltpu.async_copy(featp.at[ri_t.at[pl.ds((i + 2) * K, K)]],
                                     tgt, gs_t)

                pltpu.make_async_copy(featp.at[ri_t.at[pl.ds(i * K, K)]],
                                      cur, gs_c).wait()
                pltpu.make_async_copy(cidx.at[pl.ds((chb + i) * K, K)],
                                      cbc.at[0], is_c).wait()
                pltpu.async_copy(cur, acc_sh.at[cbc.at[0]], ss_c, add=True)

            def body(i, carry):
                for p in range(3):
                    @pl.when(lax.rem(i, 3) == p)
                    def _(i=i, p=p):
                        step(i, p)

                return carry

            lax.fori_loop(0, cpt, body, 0)
            q = cpt - 1
            pltpu.make_async_copy(rr[q % 3], acc_sh.at[cbr[q % 3].at[0]],
                                  ssr[q % 3]).wait()

        @pl.when(c == 1)
        def _():
            run(s * cpt1, cpt1)

        @pl.when(c == 0)
        def _():
            run(NS * cpt1 + s * cpt0, cpt0)

        plsc.subcore_barrier()

        pltpu.sync_copy(acc_sh.at[pl.ds(row0, wpt)], parts.at[c].at[pl.ds(row0, wpt)])

    return gather


def _combine_body(step1, last, parts_ref, f_ref, g_ref, dis_ref, c2_ref, *out_refs):
    pb = parts_ref[...]
    agg = pb[0] + pb[1]
    dis = dis_ref[...]
    zf = DTSQ * dis * agg + c2_ref[...] * f_ref[...]
    if step1:
        xn = DT * g_ref[...] + 0.5 * zf
    else:
        xn = zf - g_ref[...]
    out_refs[0][...] = xn
    if not last:
        out_refs[1][...] = dis * xn


def _combine(parts, f, g, dis, c2, br, step1, last):
    n, d = f.shape
    grid = (n // br,)
    nouts = 1 if last else 2
    return pl.pallas_call(
        functools.partial(_combine_body, step1, last),
        grid=grid,
        in_specs=[
            pl.BlockSpec((NC, br, d), lambda i: (0, i, 0)),
            pl.BlockSpec((br, d), lambda i: (i, 0)),
            pl.BlockSpec((br, d), lambda i: (i, 0)),
            pl.BlockSpec((br, 1), lambda i: (i, 0)),
            pl.BlockSpec((br, 1), lambda i: (i, 0)),
        ],
        out_specs=[pl.BlockSpec((br, d), lambda i: (i, 0))] * nouts,
        out_shape=[jax.ShapeDtypeStruct((n, d), jnp.float32)] * nouts,
    )(parts, f, g, dis, c2)


def kernel(x, edge_index, W0, b0, W1, b1):
    n, d = x.shape
    e = edge_index.shape[1]
    br = 1000

    ct_min = ceil(e / K)
    cpt1 = max(1, round(ct_min * F1 / NS))
    cpt0 = max(1, ceil((ct_min - NS * cpt1) / NS))
    e_pad = NS * (cpt0 + cpt1) * K

    wpt = ceil(ceil(n / NS) / 8) * 8
    ntrash = NS * wpt - n

    phi0, phi1 = _matmuls(x, W0, b0, W1, b1, br)
    degp, selfp, ridx, cidx = _make_stats(n, e, e_pad, ntrash)(edge_index[0], edge_index[1])
    dis, c2, featp = _prep(degp, selfp, phi0, br)

    gather = _make_gather(n, d, cpt0, cpt1)

    n_steps = ceil(TIME / DT)
    parts = gather(featp, ridx, cidx)
    xc, fp = _combine(parts, phi0, phi1, dis, c2, 2000, step1=True, last=False)
    xp = phi0
    for k in range(n_steps - 1):
        parts = gather(fp, ridx, cidx)
        if k < n_steps - 2:
            xn, fp = _combine(parts, xc, xp, dis, c2, 2000, step1=False, last=False)
        else:
            (xn,) = _combine(parts, xc, xp, dis, c2, 2000, step1=False, last=True)
        xp, xc = xc, xn
    return xc
